# baseline (speedup 1.0000x reference)
import functools

import jax
import jax.numpy as jnp
import numpy as np
from jax import lax
from jax.experimental import pallas as pl
from jax.experimental.pallas import tpu as pltpu
from jax.experimental.pallas import tpu_sc as plsc

N = 8192
K = 20
EPS = 1e-5
QB = 512
CW = 128
NC = N // CW
SENT = np.int32(0x7FFFFFFF)
NEGINF_KEY = np.int32(-0x80000000)
IDXMASK = np.int32(0x1FFF)


def _knn_body(X_ref, xqT_ref, sqT_ref, xq_ref, wy_ref, wb_ref, b_ref,
              idxT_ref, y_ref, base_ref, K3_ref, M0_ref, M1_ref, M2_ref,
              M3_ref, TK_ref):
    xqT = xqT_ref[...]
    giota = lax.broadcasted_iota(jnp.int32, (CW, QB), 0)

    def dist_chunk(c, _):
        Xc = X_ref[pl.ds(c * CW, CW), :]
        sqc = sqT_ref[pl.ds(c * CW, CW), :]
        Dc = sqc - 2.0 * jnp.dot(Xc, xqT, preferred_element_type=jnp.float32)
        bits = lax.bitcast_convert_type(Dc, jnp.int32)
        key = bits ^ (lax.shift_right_arithmetic(bits, 31) & np.int32(0x7FFFFFFF))
        key = (key & ~IDXMASK) | (giota + c * CW)
        K3_ref[c] = key
        cur = jnp.full((1, QB), NEGINF_KEY)
        for Mr in (M0_ref, M1_ref, M2_ref, M3_ref):
            cand = jnp.where(key > cur, key, SENT)
            nm = jnp.min(cand, axis=0, keepdims=True)
            Mr[c] = nm
            cur = nm
        TK_ref[c] = jnp.full((1, QB), NEGINF_KEY)
        return 0

    lax.fori_loop(0, NC, dist_chunk, 0)

    def refill_pass():
        def rc(c, _):
            keys = K3_ref[c]
            colflag = M0_ref[c] == SENT
            cur = TK_ref[c]
            for Mr in (M0_ref, M1_ref, M2_ref, M3_ref):
                cand = jnp.where(keys > cur, keys, SENT)
                nm = jnp.min(cand, axis=0, keepdims=True)
                Mr[c] = jnp.where(colflag, nm, Mr[c])
                cur = nm
            return 0
        lax.fori_loop(0, NC, rc, 0)

    siota = lax.broadcasted_iota(jnp.int32, (32, QB), 0)

    def round_body(k, carry):
        idxacc, need = carry

        @pl.when(need)
        def _():
            refill_pass()

        M0 = M0_ref[...]
        m = jnp.min(M0, axis=0, keepdims=True)
        e = (m & IDXMASK).reshape(1, QB)
        idxacc = jnp.where(siota == k, jnp.broadcast_to(e, (32, QB)), idxacc)
        mask = M0 == m
        M1 = M1_ref[...]
        M2 = M2_ref[...]
        M3 = M3_ref[...]
        M0n = jnp.where(mask, M1, M0)
        M0_ref[...] = M0n
        M1_ref[...] = jnp.where(mask, M2, M1)
        M2_ref[...] = jnp.where(mask, M3, M2)
        M3_ref[...] = jnp.where(mask, SENT, M3)
        TK_ref[...] = jnp.where(mask, jnp.broadcast_to(m, M0.shape), TK_ref[...])
        need2 = jnp.any(M0n == SENT)
        return idxacc, need2

    idxacc, _ = lax.fori_loop(
        0, 0, round_body,
        (jnp.zeros((32, QB), jnp.int32), jnp.zeros((), jnp.bool_)))
    idxT_ref[...] = idxacc

    xq = xq_ref[...]
    y_ref[...] = jnp.dot(xq, wy_ref[...], preferred_element_type=jnp.float32)
    base_ref[...] = (jnp.dot(xq, wb_ref[...], preferred_element_type=jnp.float32)
                     + b_ref[...])


def _knn_layer(xf, W, b):
    d = xf.shape[1]
    C = W.shape[0]
    dpad = max(8, d)
    if dpad != d:
        xf = jnp.pad(xf, ((0, 0), (0, dpad - d)))
    WT = W.T
    wy = WT[d:]
    wb = WT[:d] - WT[d:]
    if dpad != d:
        wy = jnp.pad(wy, ((0, dpad - d), (0, 0)))
        wb = jnp.pad(wb, ((0, dpad - d), (0, 0)))
    xT = xf.T
    sqT = jnp.sum(xf * xf, axis=1)[:, None]

    idxT, y, base = pl.pallas_call(
        _knn_body,
        grid=(N // QB,),
        in_specs=[
            pl.BlockSpec((N, dpad), lambda i: (0, 0)),
            pl.BlockSpec((dpad, QB), lambda i: (0, i)),
            pl.BlockSpec((N, 1), lambda i: (0, 0)),
            pl.BlockSpec((QB, dpad), lambda i: (i, 0)),
            pl.BlockSpec((dpad, C), lambda i: (0, 0)),
            pl.BlockSpec((dpad, C), lambda i: (0, 0)),
            pl.BlockSpec((1, C), lambda i: (0, 0)),
        ],
        out_specs=[
            pl.BlockSpec((32, QB), lambda i: (0, i)),
            pl.BlockSpec((QB, C), lambda i: (i, 0)),
            pl.BlockSpec((QB, C), lambda i: (i, 0)),
        ],
        out_shape=[
            jax.ShapeDtypeStruct((32, N), jnp.int32),
            jax.ShapeDtypeStruct((N, C), jnp.float32),
            jax.ShapeDtypeStruct((N, C), jnp.float32),
        ],
        scratch_shapes=[
            pltpu.VMEM((NC, CW, QB), jnp.int32),
            pltpu.VMEM((NC, 1, QB), jnp.int32),
            pltpu.VMEM((NC, 1, QB), jnp.int32),
            pltpu.VMEM((NC, 1, QB), jnp.int32),
            pltpu.VMEM((NC, 1, QB), jnp.int32),
            pltpu.VMEM((NC, 1, QB), jnp.int32),
        ],
    )(xf, xT, sqT, xf, wy, wb, b[None, :])
    idx = idxT[:K].T
    return idx, y, base


NW = 32
QW = N // NW


def _gather_max(y, idx_flat, base):
    C = y.shape[1]
    CQ = {128: 32, 256: 16}[C]
    mesh = plsc.VectorSubcoreMesh(core_axis_name="c", subcore_axis_name="s")

    @functools.partial(
        pl.kernel, mesh=mesh,
        out_type=jax.ShapeDtypeStruct((N, C), jnp.float32),
        scratch_types=[
            pltpu.VMEM((QW * K,), jnp.int32),
            pltpu.VMEM((CQ * K, C), jnp.float32),
            pltpu.VMEM((CQ, C), jnp.float32),
            pltpu.SemaphoreType.DMA,
        ],
    )
    def gm(y_hbm, idx_hbm, base_hbm, out_hbm, idx_v, rows_v, acc_v, sem):
        wid = lax.axis_index("s") * 2 + lax.axis_index("c")
        qbase = wid * QW
        pltpu.sync_copy(idx_hbm.at[pl.ds(qbase * K, QW * K)], idx_v)

        def chunk(ci, _):
            qoff = ci * CQ
            pltpu.async_copy(
                y_hbm.at[idx_v.at[pl.ds(qoff * K, CQ * K)]], rows_v, sem
            ).wait()
            pltpu.sync_copy(base_hbm.at[pl.ds(qbase + qoff, CQ)], acc_v)

            def per_q(q, _):
                for cs in range(C // 16):
                    sl = pl.ds(cs * 16, 16)
                    a = rows_v[q * K, sl]
                    for j in range(1, K):
                        a = jnp.maximum(a, rows_v[q * K + j, sl])
                    acc_v[q, sl] = acc_v[q, sl] + a
                return 0

            lax.fori_loop(0, CQ, per_q, 0)
            pltpu.sync_copy(acc_v, out_hbm.at[pl.ds(qbase + qoff, CQ)])
            return 0

        lax.fori_loop(0, QW // CQ, chunk, 0)

    return gm(y, idx_flat, base)


def _edge_conv(xf, W, b):
    idx, y, base = _knn_layer(xf, W, b)
    C = y.shape[1]
    if C < 128:
        y = jnp.pad(y, ((0, 0), (0, 128 - C)))
        base = jnp.pad(base, ((0, 0), (0, 128 - C)))
        return _gather_max(y, idx.reshape(-1), base)[:, :C]
    return _gather_max(y, idx.reshape(-1), base)


def _bn(x, gamma, beta):
    m = jnp.mean(x, axis=0, keepdims=True)
    v = jnp.var(x, axis=0, keepdims=True)
    return (x - m) / jnp.sqrt(v + EPS) * gamma + beta


def _tail_kernel(h_ref, w3_ref, b3_ref, ow_ref, ob_ref, out_ref):
    h = h_ref[...]
    logits = h @ w3_ref[...].T + b3_ref[...]
    out = jax.nn.sigmoid(logits @ ow_ref[...].T + ob_ref[...])
    out_ref[...] = out


def kernel(x, batch, W1, b1, W2, b2, W3, b3, geW1, geb1, geg1, gebe1, geW2, geb2, geg2, gebe2, laW1, lab1, laW2, lab2, gaW1, gab1, gaW2, gab2, fuW1, fub1, fug1, fube1, fuW2, fub2, fug2, fube2, fuW3, fub3, ordW, ordb):
    xyz = x[:, :3]
    geom = x[:, 3:]
    s1 = _edge_conv(xyz, W1, b1)
    s2 = _edge_conv(s1, W2, b2)
    s3 = _edge_conv(s2, W3, b3)
    spatial = jnp.concatenate([s1, s2, s3], axis=1)
    g = jax.nn.relu(geom @ geW1.T + geb1)
    g = _bn(g, geg1, gebe1)
    g = jax.nn.relu(g @ geW2.T + geb2)
    g = _bn(g, geg2, gebe2)
    lw = jax.nn.sigmoid(jax.nn.relu(g @ laW1.T + lab1) @ laW2.T + lab2)
    pooled = jnp.mean(g, axis=0, keepdims=True)
    gw = jax.nn.sigmoid(jax.nn.relu(pooled @ gaW1.T + gab1) @ gaW2.T + gab2)
    attended = g * (lw + gw)
    comb = jnp.concatenate([spatial, attended], axis=1)
    h = jax.nn.relu(comb @ fuW1.T + fub1)
    h = _bn(h, fug1, fube1)
    h = jax.nn.relu(h @ fuW2.T + fub2)
    h = _bn(h, fug2, fube2)
    out = pl.pallas_call(
        _tail_kernel,
        out_shape=jax.ShapeDtypeStruct((N, ordW.shape[0]), jnp.float32),
    )(h, fuW3, fub3, ordW, ordb)
    return out


# speedup vs baseline: 5.5518x; 5.5518x over previous
import functools

import jax
import jax.numpy as jnp
import numpy as np
from jax import lax
from jax.experimental import pallas as pl
from jax.experimental.pallas import tpu as pltpu
from jax.experimental.pallas import tpu_sc as plsc

N = 8192
K = 20
EPS = 1e-5
QB = 512
CW = 128
NC = N // CW
SENT = np.int32(0x7FFFFFFF)
NEGINF_KEY = np.int32(-0x80000000)
IDXMASK = np.int32(0x1FFF)


def _knn_body(X_ref, xqT_ref, sqT_ref, xq_ref, wy_ref, wb_ref, b_ref,
              idxT_ref, y_ref, base_ref, K3_ref, M0_ref, M1_ref, M2_ref,
              M3_ref, TK_ref):
    xqT = xqT_ref[...]
    giota = lax.broadcasted_iota(jnp.int32, (CW, QB), 0)

    def dist_chunk(c, _):
        Xc = X_ref[pl.ds(c * CW, CW), :]
        sqc = sqT_ref[pl.ds(c * CW, CW), :]
        Dc = sqc - 2.0 * jnp.dot(Xc, xqT, preferred_element_type=jnp.float32)
        bits = lax.bitcast_convert_type(Dc, jnp.int32)
        key = bits ^ (lax.shift_right_arithmetic(bits, 31) & np.int32(0x7FFFFFFF))
        key = (key & ~IDXMASK) | (giota + c * CW)
        K3_ref[c] = key
        cur = jnp.full((1, QB), NEGINF_KEY)
        for Mr in (M0_ref, M1_ref, M2_ref, M3_ref):
            cand = jnp.where(key > cur, key, SENT)
            nm = jnp.min(cand, axis=0, keepdims=True)
            Mr[c] = nm
            cur = nm
        TK_ref[c] = jnp.full((1, QB), NEGINF_KEY)
        return 0

    lax.fori_loop(0, NC, dist_chunk, 0)

    def refill_pass():
        def rc(c, _):
            colflag = M0_ref[c] == SENT

            @pl.when(jnp.any(colflag))
            def _():
                keys = K3_ref[c]
                cur = TK_ref[c]
                cur2 = cur
                for Mr in (M0_ref, M1_ref, M2_ref, M3_ref):
                    cand = jnp.where(keys > cur2, keys, SENT)
                    nm = jnp.min(cand, axis=0, keepdims=True)
                    Mr[c] = jnp.where(colflag, nm, Mr[c])
                    cur2 = nm
            return 0
        lax.fori_loop(0, NC, rc, 0)

    siota = lax.broadcasted_iota(jnp.int32, (32, QB), 0)

    def round_body(k, carry):
        idxacc, need = carry

        @pl.when(need)
        def _():
            refill_pass()

        M0 = M0_ref[...]
        m = jnp.min(M0, axis=0, keepdims=True)
        e = (m & IDXMASK).reshape(1, QB)
        idxacc = jnp.where(siota == k, jnp.broadcast_to(e, (32, QB)), idxacc)
        mask = M0 == m
        M1 = M1_ref[...]
        M2 = M2_ref[...]
        M3 = M3_ref[...]
        M0n = jnp.where(mask, M1, M0)
        M0_ref[...] = M0n
        M1_ref[...] = jnp.where(mask, M2, M1)
        M2_ref[...] = jnp.where(mask, M3, M2)
        M3_ref[...] = jnp.where(mask, SENT, M3)
        TK_ref[...] = jnp.where(mask, jnp.broadcast_to(m, M0.shape), TK_ref[...])
        need2 = jnp.any(M0n == SENT)
        return idxacc, need2

    idxacc, _ = lax.fori_loop(
        0, K, round_body,
        (jnp.zeros((32, QB), jnp.int32), jnp.zeros((), jnp.bool_)))
    idxT_ref[...] = idxacc

    xq = xq_ref[...]
    y_ref[...] = jnp.dot(xq, wy_ref[...], preferred_element_type=jnp.float32)
    base_ref[...] = (jnp.dot(xq, wb_ref[...], preferred_element_type=jnp.float32)
                     + b_ref[...])


def _knn_layer(xf, W, b):
    d = xf.shape[1]
    C = W.shape[0]
    dpad = max(8, d)
    if dpad != d:
        xf = jnp.pad(xf, ((0, 0), (0, dpad - d)))
    WT = W.T
    wy = WT[d:]
    wb = WT[:d] - WT[d:]
    if dpad != d:
        wy = jnp.pad(wy, ((0, dpad - d), (0, 0)))
        wb = jnp.pad(wb, ((0, dpad - d), (0, 0)))
    xT = xf.T
    sqT = jnp.sum(xf * xf, axis=1)[:, None]

    idxT, y, base = pl.pallas_call(
        _knn_body,
        grid=(N // QB,),
        in_specs=[
            pl.BlockSpec((N, dpad), lambda i: (0, 0)),
            pl.BlockSpec((dpad, QB), lambda i: (0, i)),
            pl.BlockSpec((N, 1), lambda i: (0, 0)),
            pl.BlockSpec((QB, dpad), lambda i: (i, 0)),
            pl.BlockSpec((dpad, C), lambda i: (0, 0)),
            pl.BlockSpec((dpad, C), lambda i: (0, 0)),
            pl.BlockSpec((1, C), lambda i: (0, 0)),
        ],
        out_specs=[
            pl.BlockSpec((32, QB), lambda i: (0, i)),
            pl.BlockSpec((QB, C), lambda i: (i, 0)),
            pl.BlockSpec((QB, C), lambda i: (i, 0)),
        ],
        out_shape=[
            jax.ShapeDtypeStruct((32, N), jnp.int32),
            jax.ShapeDtypeStruct((N, C), jnp.float32),
            jax.ShapeDtypeStruct((N, C), jnp.float32),
        ],
        scratch_shapes=[
            pltpu.VMEM((NC, CW, QB), jnp.int32),
            pltpu.VMEM((NC, 1, QB), jnp.int32),
            pltpu.VMEM((NC, 1, QB), jnp.int32),
            pltpu.VMEM((NC, 1, QB), jnp.int32),
            pltpu.VMEM((NC, 1, QB), jnp.int32),
            pltpu.VMEM((NC, 1, QB), jnp.int32),
        ],
    )(xf, xT, sqT, xf, wy, wb, b[None, :])
    idx = idxT[:K].T
    return idx, y, base


NW = 32
QW = N // NW


def _gather_max(y, idx_flat, base):
    C = y.shape[1]
    CQ = {128: 32, 256: 16}[C]
    mesh = plsc.VectorSubcoreMesh(core_axis_name="c", subcore_axis_name="s")

    @functools.partial(
        pl.kernel, mesh=mesh,
        out_type=jax.ShapeDtypeStruct((N, C), jnp.float32),
        scratch_types=[
            pltpu.VMEM((QW * K,), jnp.int32),
            pltpu.VMEM((CQ * K, C), jnp.float32),
            pltpu.VMEM((CQ, C), jnp.float32),
            pltpu.SemaphoreType.DMA,
        ],
    )
    def gm(y_hbm, idx_hbm, base_hbm, out_hbm, idx_v, rows_v, acc_v, sem):
        wid = lax.axis_index("s") * 2 + lax.axis_index("c")
        qbase = wid * QW
        pltpu.sync_copy(idx_hbm.at[pl.ds(qbase * K, QW * K)], idx_v)

        def chunk(ci, _):
            qoff = ci * CQ
            pltpu.async_copy(
                y_hbm.at[idx_v.at[pl.ds(qoff * K, CQ * K)]], rows_v, sem
            ).wait()
            pltpu.sync_copy(base_hbm.at[pl.ds(qbase + qoff, CQ)], acc_v)

            def per_q(q, _):
                for cs in range(C // 16):
                    sl = pl.ds(cs * 16, 16)
                    a = rows_v[q * K, sl]
                    for j in range(1, K):
                        a = jnp.maximum(a, rows_v[q * K + j, sl])
                    acc_v[q, sl] = acc_v[q, sl] + a
                return 0

            lax.fori_loop(0, CQ, per_q, 0)
            pltpu.sync_copy(acc_v, out_hbm.at[pl.ds(qbase + qoff, CQ)])
            return 0

        lax.fori_loop(0, QW // CQ, chunk, 0)

    return gm(y, idx_flat, base)


def _edge_conv(xf, W, b):
    idx, y, base = _knn_layer(xf, W, b)
    C = y.shape[1]
    if C < 128:
        y = jnp.pad(y, ((0, 0), (0, 128 - C)))
        base = jnp.pad(base, ((0, 0), (0, 128 - C)))
        return _gather_max(y, idx.reshape(-1), base)[:, :C]
    return _gather_max(y, idx.reshape(-1), base)


def _bn(x, gamma, beta):
    m = jnp.mean(x, axis=0, keepdims=True)
    v = jnp.var(x, axis=0, keepdims=True)
    return (x - m) / jnp.sqrt(v + EPS) * gamma + beta


def _tail_kernel(h_ref, w3_ref, b3_ref, ow_ref, ob_ref, out_ref):
    h = h_ref[...]
    logits = h @ w3_ref[...].T + b3_ref[...]
    out = jax.nn.sigmoid(logits @ ow_ref[...].T + ob_ref[...])
    out_ref[...] = out


def kernel(x, batch, W1, b1, W2, b2, W3, b3, geW1, geb1, geg1, gebe1, geW2, geb2, geg2, gebe2, laW1, lab1, laW2, lab2, gaW1, gab1, gaW2, gab2, fuW1, fub1, fug1, fube1, fuW2, fub2, fug2, fube2, fuW3, fub3, ordW, ordb):
    xyz = x[:, :3]
    geom = x[:, 3:]
    s1 = _edge_conv(xyz, W1, b1)
    s2 = _edge_conv(s1, W2, b2)
    s3 = _edge_conv(s2, W3, b3)
    spatial = jnp.concatenate([s1, s2, s3], axis=1)
    g = jax.nn.relu(geom @ geW1.T + geb1)
    g = _bn(g, geg1, gebe1)
    g = jax.nn.relu(g @ geW2.T + geb2)
    g = _bn(g, geg2, gebe2)
    lw = jax.nn.sigmoid(jax.nn.relu(g @ laW1.T + lab1) @ laW2.T + lab2)
    pooled = jnp.mean(g, axis=0, keepdims=True)
    gw = jax.nn.sigmoid(jax.nn.relu(pooled @ gaW1.T + gab1) @ gaW2.T + gab2)
    attended = g * (lw + gw)
    comb = jnp.concatenate([spatial, attended], axis=1)
    h = jax.nn.relu(comb @ fuW1.T + fub1)
    h = _bn(h, fug1, fube1)
    h = jax.nn.relu(h @ fuW2.T + fub2)
    h = _bn(h, fug2, fube2)
    out = pl.pallas_call(
        _tail_kernel,
        out_shape=jax.ShapeDtypeStruct((N, ordW.shape[0]), jnp.float32),
    )(h, fuW3, fub3, ordW, ordb)
    return out


# speedup vs baseline: 8.6552x; 1.5590x over previous
import functools

import jax
import jax.numpy as jnp
import numpy as np
from jax import lax
from jax.experimental import pallas as pl
from jax.experimental.pallas import tpu as pltpu
from jax.experimental.pallas import tpu_sc as plsc

N = 8192
K = 20
EPS = 1e-5
QB = 512
CW = 128
NC = N // CW
SENT = np.int32(0x7FFFFFFF)
NEGINF_KEY = np.int32(-0x80000000)
IDXMASK = np.int32(0x1FFF)


def _knn_body(X_ref, xqT_ref, sqT_ref, xq_ref, wy_ref, wb_ref, b_ref,
              idxT_ref, y_ref, base_ref, K3_ref, M0_ref, M1_ref, M2_ref,
              M3_ref, M4_ref, TK_ref):
    xqT = xqT_ref[...]
    giota = lax.broadcasted_iota(jnp.int32, (CW, QB), 0)

    def dist_chunk(c, _):
        Xc = X_ref[pl.ds(c * CW, CW), :]
        sqc = sqT_ref[pl.ds(c * CW, CW), :]
        Dc = sqc - 2.0 * jnp.dot(Xc, xqT, preferred_element_type=jnp.float32)
        bits = lax.bitcast_convert_type(Dc, jnp.int32)
        key = bits ^ (lax.shift_right_arithmetic(bits, 31) & np.int32(0x7FFFFFFF))
        key = (key & ~IDXMASK) | (giota + c * CW)
        K3_ref[c] = key
        cur = jnp.full((1, QB), NEGINF_KEY)
        for Mr in (M0_ref, M1_ref, M2_ref, M3_ref, M4_ref):
            cand = jnp.where(key > cur, key, SENT)
            nm = jnp.min(cand, axis=0, keepdims=True)
            Mr[c] = nm
            cur = nm
        TK_ref[c] = jnp.full((1, QB), NEGINF_KEY)
        return 0

    lax.fori_loop(0, NC, dist_chunk, 0)

    def refill_pass():
        def rc(c, _):
            colflag = M0_ref[c] == SENT

            @pl.when(jnp.any(colflag))
            def _():
                keys = K3_ref[c]
                cur = TK_ref[c]
                cur2 = cur
                for Mr in (M0_ref, M1_ref, M2_ref, M3_ref, M4_ref):
                    cand = jnp.where(keys > cur2, keys, SENT)
                    nm = jnp.min(cand, axis=0, keepdims=True)
                    Mr[c] = jnp.where(colflag, nm, Mr[c])
                    cur2 = nm
            return 0
        lax.fori_loop(0, NC, rc, 0)

    siota = lax.broadcasted_iota(jnp.int32, (32, QB), 0)

    def round_body(k, carry):
        idxacc, need = carry

        @pl.when(need)
        def _():
            refill_pass()

        M0 = M0_ref[...]
        m = jnp.min(M0, axis=0, keepdims=True)
        e = (m & IDXMASK).reshape(1, QB)
        idxacc = jnp.where(siota == k, jnp.broadcast_to(e, (32, QB)), idxacc)
        mask = M0 == m
        M1 = M1_ref[...]
        M2 = M2_ref[...]
        M3 = M3_ref[...]
        M4 = M4_ref[...]
        M0n = jnp.where(mask, M1, M0)
        M0_ref[...] = M0n
        M1_ref[...] = jnp.where(mask, M2, M1)
        M2_ref[...] = jnp.where(mask, M3, M2)
        M3_ref[...] = jnp.where(mask, M4, M3)
        M4_ref[...] = jnp.where(mask, SENT, M4)
        TK_ref[...] = jnp.where(mask, jnp.broadcast_to(m, M0.shape), TK_ref[...])
        need2 = jnp.any(M0n == SENT)
        return idxacc, need2

    idxacc, _ = lax.fori_loop(
        0, K, round_body,
        (jnp.zeros((32, QB), jnp.int32), jnp.zeros((), jnp.bool_)))
    idxT_ref[...] = idxacc

    xq = xq_ref[...]
    y_ref[...] = jnp.dot(xq, wy_ref[...], preferred_element_type=jnp.float32)
    base_ref[...] = (jnp.dot(xq, wb_ref[...], preferred_element_type=jnp.float32)
                     + b_ref[...])


def _knn_layer(xf, W, b):
    d = xf.shape[1]
    C = W.shape[0]
    dpad = max(8, d)
    if dpad != d:
        xf = jnp.pad(xf, ((0, 0), (0, dpad - d)))
    WT = W.T
    wy = WT[d:]
    wb = WT[:d] - WT[d:]
    if dpad != d:
        wy = jnp.pad(wy, ((0, dpad - d), (0, 0)))
        wb = jnp.pad(wb, ((0, dpad - d), (0, 0)))
    xT = xf.T
    sqT = jnp.sum(xf * xf, axis=1)[:, None]

    idxT, y, base = pl.pallas_call(
        _knn_body,
        grid=(N // QB,),
        in_specs=[
            pl.BlockSpec((N, dpad), lambda i: (0, 0)),
            pl.BlockSpec((dpad, QB), lambda i: (0, i)),
            pl.BlockSpec((N, 1), lambda i: (0, 0)),
            pl.BlockSpec((QB, dpad), lambda i: (i, 0)),
            pl.BlockSpec((dpad, C), lambda i: (0, 0)),
            pl.BlockSpec((dpad, C), lambda i: (0, 0)),
            pl.BlockSpec((1, C), lambda i: (0, 0)),
        ],
        out_specs=[
            pl.BlockSpec((32, QB), lambda i: (0, i)),
            pl.BlockSpec((QB, C), lambda i: (i, 0)),
            pl.BlockSpec((QB, C), lambda i: (i, 0)),
        ],
        out_shape=[
            jax.ShapeDtypeStruct((32, N), jnp.int32),
            jax.ShapeDtypeStruct((N, C), jnp.float32),
            jax.ShapeDtypeStruct((N, C), jnp.float32),
        ],
        scratch_shapes=[
            pltpu.VMEM((NC, CW, QB), jnp.int32),
            pltpu.VMEM((NC, 1, QB), jnp.int32),
            pltpu.VMEM((NC, 1, QB), jnp.int32),
            pltpu.VMEM((NC, 1, QB), jnp.int32),
            pltpu.VMEM((NC, 1, QB), jnp.int32),
            pltpu.VMEM((NC, 1, QB), jnp.int32),
            pltpu.VMEM((NC, 1, QB), jnp.int32),
        ],
    )(xf, xT, sqT, xf, wy, wb, b[None, :])
    idx = idxT[:K].T
    return idx, y, base


NW = 32
QW = N // NW


def _gather_max(y, idx_flat, base):
    C = y.shape[1]
    CQ = {128: 32, 256: 16}[C]
    mesh = plsc.VectorSubcoreMesh(core_axis_name="c", subcore_axis_name="s")

    @functools.partial(
        pl.kernel, mesh=mesh,
        out_type=jax.ShapeDtypeStruct((N, C), jnp.float32),
        scratch_types=[
            pltpu.VMEM((QW * K,), jnp.int32),
            pltpu.VMEM((CQ * K, C), jnp.float32),
            pltpu.VMEM((CQ, C), jnp.float32),
            pltpu.SemaphoreType.DMA,
        ],
    )
    def gm(y_hbm, idx_hbm, base_hbm, out_hbm, idx_v, rows_v, acc_v, sem):
        wid = lax.axis_index("s") * 2 + lax.axis_index("c")
        qbase = wid * QW
        pltpu.sync_copy(idx_hbm.at[pl.ds(qbase * K, QW * K)], idx_v)

        def chunk(ci, _):
            qoff = ci * CQ
            pltpu.async_copy(
                y_hbm.at[idx_v.at[pl.ds(qoff * K, CQ * K)]], rows_v, sem
            ).wait()
            pltpu.sync_copy(base_hbm.at[pl.ds(qbase + qoff, CQ)], acc_v)

            def per_q(q, _):
                for cs in range(C // 16):
                    sl = pl.ds(cs * 16, 16)
                    a = rows_v[q * K, sl]
                    for j in range(1, K):
                        a = jnp.maximum(a, rows_v[q * K + j, sl])
                    acc_v[q, sl] = acc_v[q, sl] + a
                return 0

            lax.fori_loop(0, CQ, per_q, 0)
            pltpu.sync_copy(acc_v, out_hbm.at[pl.ds(qbase + qoff, CQ)])
            return 0

        lax.fori_loop(0, QW // CQ, chunk, 0)

    return gm(y, idx_flat, base)


def _edge_conv(xf, W, b):
    idx, y, base = _knn_layer(xf, W, b)
    C = y.shape[1]
    if C < 128:
        y = jnp.pad(y, ((0, 0), (0, 128 - C)))
        base = jnp.pad(base, ((0, 0), (0, 128 - C)))
        return _gather_max(y, idx.reshape(-1), base)[:, :C]
    return _gather_max(y, idx.reshape(-1), base)


def _bn(x, gamma, beta):
    m = jnp.mean(x, axis=0, keepdims=True)
    v = jnp.var(x, axis=0, keepdims=True)
    return (x - m) / jnp.sqrt(v + EPS) * gamma + beta


def _tail_kernel(h_ref, w3_ref, b3_ref, ow_ref, ob_ref, out_ref):
    h = h_ref[...]
    logits = h @ w3_ref[...].T + b3_ref[...]
    out = jax.nn.sigmoid(logits @ ow_ref[...].T + ob_ref[...])
    out_ref[...] = out


def kernel(x, batch, W1, b1, W2, b2, W3, b3, geW1, geb1, geg1, gebe1, geW2, geb2, geg2, gebe2, laW1, lab1, laW2, lab2, gaW1, gab1, gaW2, gab2, fuW1, fub1, fug1, fube1, fuW2, fub2, fug2, fube2, fuW3, fub3, ordW, ordb):
    xyz = x[:, :3]
    geom = x[:, 3:]
    s1 = _edge_conv(xyz, W1, b1)
    s2 = _edge_conv(s1, W2, b2)
    s3 = _edge_conv(s2, W3, b3)
    spatial = jnp.concatenate([s1, s2, s3], axis=1)
    g = jax.nn.relu(geom @ geW1.T + geb1)
    g = _bn(g, geg1, gebe1)
    g = jax.nn.relu(g @ geW2.T + geb2)
    g = _bn(g, geg2, gebe2)
    lw = jax.nn.sigmoid(jax.nn.relu(g @ laW1.T + lab1) @ laW2.T + lab2)
    pooled = jnp.mean(g, axis=0, keepdims=True)
    gw = jax.nn.sigmoid(jax.nn.relu(pooled @ gaW1.T + gab1) @ gaW2.T + gab2)
    attended = g * (lw + gw)
    comb = jnp.concatenate([spatial, attended], axis=1)
    h = jax.nn.relu(comb @ fuW1.T + fub1)
    h = _bn(h, fug1, fube1)
    h = jax.nn.relu(h @ fuW2.T + fub2)
    h = _bn(h, fug2, fube2)
    out = pl.pallas_call(
        _tail_kernel,
        out_shape=jax.ShapeDtypeStruct((N, ordW.shape[0]), jnp.float32),
    )(h, fuW3, fub3, ordW, ordb)
    return out
